# Initial kernel scaffold; baseline (speedup 1.0000x reference)
import jax
import jax.numpy as jnp
from jax.experimental import pallas as pl


def kernel(x, W1, b1, W2, b2, Wg, bg):
    raise NotImplementedError("write your pallas kernel here")


# speedup vs baseline: 1.2449x; 1.2449x over previous
import functools
import jax
import jax.numpy as jnp
from jax.experimental import pallas as pl
from jax.experimental.pallas import tpu as pltpu

DIM = 768
HID = 1536
E = 8
K = 2
N = 2048

TB = 256


def _moe_dense_kernel(x_ref, w1_ref, b1_ref, w2_ref, b2_ref, wg_ref, bg_ref,
                      out_ref):
    xb = x_ref[...]
    logits = jnp.dot(xb, wg_ref[...], preferred_element_type=jnp.float32)
    logits = logits + bg_ref[...]
    m = jnp.max(logits, axis=-1, keepdims=True)
    p = jnp.exp(logits - m)
    p = p / jnp.sum(p, axis=-1, keepdims=True)
    cols = jax.lax.broadcasted_iota(jnp.int32, p.shape, 1)
    i1 = jnp.argmax(p, axis=-1)
    is1 = cols == i1[:, None]
    p1 = jnp.max(p, axis=-1, keepdims=True)
    pm = jnp.where(is1, -1.0, p)
    i2 = jnp.argmax(pm, axis=-1)
    is2 = cols == i2[:, None]
    p2 = jnp.max(pm, axis=-1, keepdims=True)
    gates = jnp.where(is1, p1, 0.0) + jnp.where(is2, p2, 0.0)

    xbf = xb.astype(jnp.bfloat16)
    acc = jnp.zeros((TB, DIM), jnp.float32)
    for e in range(E):
        h = jnp.dot(xbf, w1_ref[e], preferred_element_type=jnp.float32)
        h = jnp.maximum(h + b1_ref[e][None, :], 0.0)
        y = jnp.dot(h.astype(jnp.bfloat16), w2_ref[e],
                    preferred_element_type=jnp.float32)
        y = y + b2_ref[e][None, :]
        acc = acc + gates[:, e][:, None] * y
    out_ref[...] = acc


@jax.jit
def kernel(x, W1, b1, W2, b2, Wg, bg):
    w1b = W1.astype(jnp.bfloat16)
    w2b = W2.astype(jnp.bfloat16)
    grid = (N // TB,)
    return pl.pallas_call(
        _moe_dense_kernel,
        grid=grid,
        in_specs=[
            pl.BlockSpec((TB, DIM), lambda i: (i, 0)),
            pl.BlockSpec((E, DIM, HID), lambda i: (0, 0, 0)),
            pl.BlockSpec((E, HID), lambda i: (0, 0)),
            pl.BlockSpec((E, HID, DIM), lambda i: (0, 0, 0)),
            pl.BlockSpec((E, DIM), lambda i: (0, 0)),
            pl.BlockSpec((DIM, E), lambda i: (0, 0)),
            pl.BlockSpec((1, E), lambda i: (0, 0)),
        ],
        out_specs=pl.BlockSpec((TB, DIM), lambda i: (i, 0)),
        out_shape=jax.ShapeDtypeStruct((N, DIM), jnp.float32),
    )(x, w1b, b1, w2b, b2, Wg, bg.reshape(1, E))
